# baseline (speedup 1.0000x reference)
import functools

import jax
import jax.numpy as jnp
from jax import lax
from jax.experimental import pallas as pl
from jax.experimental.pallas import tpu as pltpu
from jax.experimental.pallas import tpu_sc as plsc

ZMAX = 100
NUM_FEATURES = 128
N_ATOMS = 16384

_NC = 2
_NS = 16
_NW = _NC * _NS
_B_PER_W = N_ATOMS // _NW
_LANES = 16

_CHUNK = 128
_N_CHUNK = _B_PER_W // _CHUNK
_N_GATHER = 2


def _table_body(ee_ref, cw_ref, ec_ref, out_ref):
    proj = lax.dot_general(
        ec_ref[...], cw_ref[...],
        dimension_numbers=(((1,), (1,)), ((), ())),
        preferred_element_type=jnp.float32,
    )
    out_ref[...] = ee_ref[...] + proj


_build_table = pl.pallas_call(
    _table_body,
    out_shape=jax.ShapeDtypeStruct((ZMAX, NUM_FEATURES), jnp.float32),
)

_sc_mesh = plsc.VectorSubcoreMesh(core_axis_name="c", subcore_axis_name="s")


@functools.partial(
    pl.kernel,
    mesh=_sc_mesh,
    out_type=jax.ShapeDtypeStruct((N_ATOMS, NUM_FEATURES), jnp.float32),
    scratch_types=[
        pltpu.VMEM((ZMAX, NUM_FEATURES), jnp.float32),
        pltpu.VMEM((_N_GATHER, _CHUNK, NUM_FEATURES), jnp.float32),
        pltpu.VMEM((_N_CHUNK - _N_GATHER, _CHUNK, NUM_FEATURES), jnp.float32),
        pltpu.VMEM((_B_PER_W,), jnp.int32),
        pltpu.SemaphoreType.DMA((_N_CHUNK,)),
        pltpu.SemaphoreType.DMA((_N_GATHER,)),
    ],
)
def _sc_lookup(table_hbm, idx_hbm, out_hbm, table_v, rows_g, rows_b, idx_v,
               ssem, gsem):
    wid = lax.axis_index("s") * _NC + lax.axis_index("c")
    base = wid * _B_PER_W
    pltpu.sync_copy(idx_hbm.at[pl.ds(base, _B_PER_W)], idx_v)
    pltpu.sync_copy(table_hbm, table_v)

    gathers = [
        pltpu.async_copy(
            table_hbm.at[idx_v.at[pl.ds(c * _CHUNK, _CHUNK)]],
            rows_g.at[c], gsem.at[c])
        for c in range(_N_GATHER)
    ]
    scatters = []
    for c in range(_N_GATHER, _N_CHUNK):
        buf = rows_b.at[c - _N_GATHER]

        @plsc.parallel_loop(0, _CHUNK, step=_LANES, unroll=2)
        def _(r0):
            zv = idx_v[pl.ds(c * _CHUNK + r0, _LANES)]
            for j in range(_LANES):
                z = zv[j]
                for k in range(NUM_FEATURES // _LANES):
                    buf[r0 + j, pl.ds(k * _LANES, _LANES)] = (
                        table_v[z, pl.ds(k * _LANES, _LANES)])

        scatters.append(pltpu.async_copy(
            buf, out_hbm.at[pl.ds(base + c * _CHUNK, _CHUNK)], ssem.at[c]))
    for c in range(_N_GATHER):
        gathers[c].wait()
        scatters.append(pltpu.async_copy(
            rows_g.at[c], out_hbm.at[pl.ds(base + c * _CHUNK, _CHUNK)],
            ssem.at[c]))
    for s in scatters:
        s.wait()


def kernel(Z, element_embedding, config_weight, electron_config):
    table = _build_table(element_embedding, config_weight, electron_config)
    return _sc_lookup(table, Z.astype(jnp.int32))


# speedup vs baseline: 1.5852x; 1.5852x over previous
import functools

import jax
import jax.numpy as jnp
from jax import lax
from jax.experimental import pallas as pl
from jax.experimental.pallas import tpu as pltpu
from jax.experimental.pallas import tpu_sc as plsc

ZMAX = 100
NUM_FEATURES = 128
N_ATOMS = 16384

_NC = 2
_NS = 16
_NW = _NC * _NS
_B_PER_W = N_ATOMS // _NW
_LANES = 16

_CHUNK = 128
_N_CHUNK = _B_PER_W // _CHUNK
_N_GATHER = 2


def _table_body(ee_ref, cw_ref, ec_ref, out_ref):
    proj = lax.dot_general(
        ec_ref[...], cw_ref[...],
        dimension_numbers=(((1,), (1,)), ((), ())),
        preferred_element_type=jnp.float32,
    )
    out_ref[...] = ee_ref[...] + proj


_build_table = pl.pallas_call(
    _table_body,
    out_shape=jax.ShapeDtypeStruct((ZMAX, NUM_FEATURES), jnp.float32),
)

_sc_mesh = plsc.VectorSubcoreMesh(core_axis_name="c", subcore_axis_name="s")


@functools.partial(
    pl.kernel,
    mesh=_sc_mesh,
    out_type=jax.ShapeDtypeStruct((N_ATOMS, NUM_FEATURES), jnp.float32),
    scratch_types=[
        pltpu.VMEM_SHARED((ZMAX, NUM_FEATURES), jnp.float32),
        pltpu.VMEM((_N_CHUNK, _CHUNK, NUM_FEATURES), jnp.float32),
        pltpu.VMEM((_B_PER_W,), jnp.int32),
        pltpu.SemaphoreType.DMA((_N_CHUNK,)),
        pltpu.SemaphoreType.DMA((_N_CHUNK,)),
    ],
)
def _sc_lookup(table_hbm, idx_hbm, out_hbm, table_sh, rows_v, idx_v, ssem,
               gsem):
    sid = lax.axis_index("s")
    wid = sid * _NC + lax.axis_index("c")
    base = wid * _B_PER_W
    pltpu.sync_copy(idx_hbm.at[pl.ds(base, _B_PER_W)], idx_v)

    @pl.when(sid == 0)
    def _():
        pltpu.sync_copy(table_hbm, table_sh)

    plsc.subcore_barrier()

    gathers = [
        pltpu.async_copy(
            table_sh.at[idx_v.at[pl.ds(c * _CHUNK, _CHUNK)]],
            rows_v.at[c], gsem.at[c])
        for c in range(_N_CHUNK)
    ]
    scatters = []
    for c in range(_N_CHUNK):
        gathers[c].wait()
        scatters.append(pltpu.async_copy(
            rows_v.at[c], out_hbm.at[pl.ds(base + c * _CHUNK, _CHUNK)],
            ssem.at[c]))
    for s in scatters:
        s.wait()


def kernel(Z, element_embedding, config_weight, electron_config):
    table = _build_table(element_embedding, config_weight, electron_config)
    return _sc_lookup(table, Z.astype(jnp.int32))
